# initial kernel scaffold (unmeasured)
import jax
import jax.numpy as jnp
from jax import lax
from jax.experimental import pallas as pl
from jax.experimental.pallas import tpu as pltpu

N_DEV = 32


def _ring_allreduce(y):
    M, N = y.shape
    CH = M // N_DEV

    def body(y_ref, out_ref, send_bufs, recv_bufs, send_sems, recv_sems,
             load_sems, store_sem, credit_sem):
        r = lax.axis_index("i")
        left = lax.rem(r - 1 + N_DEV, N_DEV)
        right = lax.rem(r + 1, N_DEV)

        barrier_sem = pltpu.get_barrier_semaphore()
        for nbr in (left, right):
            pl.semaphore_signal(
                barrier_sem, inc=1,
                device_id=(nbr,), device_id_type=pl.DeviceIdType.MESH,
            )
        pl.semaphore_wait(barrier_sem, 2)

        def load_chunk(idx, vbuf, sem):
            cp = pltpu.make_async_copy(
                y_ref.at[pl.ds(idx * CH, CH), :], vbuf, sem)
            cp.start()
            return cp

        def store_chunk(vbuf, idx):
            cp = pltpu.make_async_copy(
                vbuf, out_ref.at[pl.ds(idx * CH, CH), :], store_sem)
            cp.start()
            return cp

        def hop(g, src_slot):
            if g >= 2:
                pl.semaphore_wait(credit_sem, 1)
            rdma = pltpu.make_async_remote_copy(
                src_ref=send_bufs.at[src_slot],
                dst_ref=recv_bufs.at[g % 2],
                send_sem=send_sems.at[g % 2],
                recv_sem=recv_sems.at[g % 2],
                device_id=(right,),
                device_id_type=pl.DeviceIdType.MESH,
            )
            rdma.start()
            return rdma

        def credit_left():
            pl.semaphore_signal(
                credit_sem, inc=1,
                device_id=(left,), device_id_type=pl.DeviceIdType.MESH,
            )

        load_chunk(r, send_bufs.at[0], load_sems.at[0]).wait()

        for g in range(N_DEV - 1):
            slot = g % 2
            nxt = (g + 1) % 2
            rdma = hop(g, slot)
            idx = lax.rem(r - (g + 1) + N_DEV, N_DEV)
            cp = load_chunk(idx, send_bufs.at[nxt], load_sems.at[nxt])
            cp.wait()
            rdma.wait()
            send_bufs[nxt] = send_bufs[nxt] + recv_bufs[slot]
            credit_left()

        own_slot = (N_DEV - 1) % 2
        own = lax.rem(r + 1, N_DEV)
        store_chunk(send_bufs.at[own_slot], own).wait()

        for t in range(N_DEV - 1):
            g = (N_DEV - 1) + t
            slot = g % 2
            nxt = (g + 1) % 2
            rdma = hop(g, slot)
            rdma.wait()
            idx = lax.rem(r - t + N_DEV, N_DEV)
            st = store_chunk(recv_bufs.at[slot], idx)
            if t < N_DEV - 2:
                send_bufs[nxt] = recv_bufs[slot]
            st.wait()
            credit_left()

        pl.semaphore_wait(credit_sem, 2)

    return pl.pallas_call(
        body,
        out_shape=jax.ShapeDtypeStruct((M, N), jnp.float32),
        in_specs=[pl.BlockSpec(memory_space=pltpu.ANY)],
        out_specs=pl.BlockSpec(memory_space=pltpu.ANY),
        scratch_shapes=[
            pltpu.VMEM((2, CH, N), jnp.float32),
            pltpu.VMEM((2, CH, N), jnp.float32),
            pltpu.SemaphoreType.DMA((2,)),
            pltpu.SemaphoreType.DMA((2,)),
            pltpu.SemaphoreType.DMA((2,)),
            pltpu.SemaphoreType.DMA,
            pltpu.SemaphoreType.REGULAR,
        ],
        compiler_params=pltpu.CompilerParams(collective_id=0),
    )(y)


def kernel(x, w_mat):
    y = jnp.dot(x, w_mat, preferred_element_type=jnp.float32)
    y = _ring_allreduce(y)
    amax = jnp.max(jnp.abs(y))
    scale = amax / 448.0
    q = (y / scale).astype(jnp.float8_e4m3fn)
    return q.astype(jnp.float32) * scale


# baseline (device time: 3318618 ns/iter reference)
import jax
import jax.numpy as jnp
from jax import lax
from jax.experimental import pallas as pl
from jax.experimental.pallas import tpu as pltpu

N_DEV = 32


def _ring_allreduce(y):
    M, N = y.shape
    CH = M // N_DEV

    def body(y_ref, out_ref, send_bufs, recv_bufs, send_sems, recv_sems,
             load_sems, store_sem, credit_sem):
        r = lax.axis_index("i")
        left = lax.rem(r - 1 + N_DEV, N_DEV)
        right = lax.rem(r + 1, N_DEV)

        barrier_sem = pltpu.get_barrier_semaphore()
        for nbr in (left, right):
            pl.semaphore_signal(
                barrier_sem, inc=1,
                device_id=(nbr,), device_id_type=pl.DeviceIdType.MESH,
            )
        pl.semaphore_wait(barrier_sem, 2)

        def load_chunk(idx, vbuf, sem):
            cp = pltpu.make_async_copy(
                y_ref.at[pl.ds(idx * CH, CH), :], vbuf, sem)
            cp.start()
            return cp

        def store_chunk(vbuf, idx):
            cp = pltpu.make_async_copy(
                vbuf, out_ref.at[pl.ds(idx * CH, CH), :], store_sem)
            cp.start()
            return cp

        def hop(g, src_slot):
            if g >= 2:
                pl.semaphore_wait(credit_sem, 1)
            rdma = pltpu.make_async_remote_copy(
                src_ref=send_bufs.at[src_slot],
                dst_ref=recv_bufs.at[g % 2],
                send_sem=send_sems.at[g % 2],
                recv_sem=recv_sems.at[g % 2],
                device_id=(right,),
                device_id_type=pl.DeviceIdType.MESH,
            )
            rdma.start()
            return rdma

        def credit_left():
            pl.semaphore_signal(
                credit_sem, inc=1,
                device_id=(left,), device_id_type=pl.DeviceIdType.MESH,
            )

        load_chunk(r, send_bufs.at[0], load_sems.at[0]).wait()

        for g in range(N_DEV - 1):
            slot = g % 2
            nxt = (g + 1) % 2
            rdma = hop(g, slot)
            idx = lax.rem(r - (g + 1) + N_DEV, N_DEV)
            cp = load_chunk(idx, send_bufs.at[nxt], load_sems.at[nxt])
            cp.wait()
            rdma.wait()
            send_bufs[nxt] = send_bufs[nxt] + recv_bufs[slot]
            credit_left()

        own_slot = (N_DEV - 1) % 2
        own = lax.rem(r + 1, N_DEV)
        store_chunk(send_bufs.at[own_slot], own).wait()

        for t in range(N_DEV - 1):
            g = (N_DEV - 1) + t
            slot = g % 2
            nxt = (g + 1) % 2
            rdma = hop(g, slot)
            rdma.wait()
            idx = lax.rem(r - t + N_DEV, N_DEV)
            st = store_chunk(recv_bufs.at[slot], idx)
            if t < N_DEV - 2:
                send_bufs[nxt] = recv_bufs[slot]
            st.wait()
            credit_left()

        pl.semaphore_wait(credit_sem, 2)

    return pl.pallas_call(
        body,
        out_shape=jax.ShapeDtypeStruct((M, N), jnp.float32),
        in_specs=[pl.BlockSpec(memory_space=pl.ANY)],
        out_specs=pl.BlockSpec(memory_space=pl.ANY),
        scratch_shapes=[
            pltpu.VMEM((2, CH, N), jnp.float32),
            pltpu.VMEM((2, CH, N), jnp.float32),
            pltpu.SemaphoreType.DMA((2,)),
            pltpu.SemaphoreType.DMA((2,)),
            pltpu.SemaphoreType.DMA((2,)),
            pltpu.SemaphoreType.DMA,
            pltpu.SemaphoreType.REGULAR,
        ],
        compiler_params=pltpu.CompilerParams(collective_id=0),
    )(y)


def kernel(x, w_mat):
    y = jnp.dot(x, w_mat, preferred_element_type=jnp.float32,
                precision=lax.Precision.HIGHEST)
    y = _ring_allreduce(y)
    amax = jnp.max(jnp.abs(y))
    scale = amax / 448.0
    v = y / scale
    a = jnp.abs(v)
    _, e = jnp.frexp(a)
    q = jnp.where(a >= 2.0 ** -6,
                  jnp.ldexp(jnp.float32(1.0), e - 4),
                  jnp.float32(2.0 ** -9))
    snapped = jnp.minimum(jnp.round(a / q) * q, 448.0)
    return jnp.sign(v) * snapped * scale


# device time: 1920796 ns/iter; 1.7277x vs baseline; 1.7277x over previous
import jax
import jax.numpy as jnp
from jax import lax
from jax.experimental import pallas as pl
from jax.experimental.pallas import tpu as pltpu

N_DEV = 32


def _hamiltonian_cycle():
    path = []
    for y in range(4):
        zs = range(4) if y % 2 == 0 else range(3, -1, -1)
        path.extend((y, z) for z in zs)
    cyc = [(0, y, z) for (y, z) in path]
    cyc += [(1, y, z) for (y, z) in reversed(path)]

    def midx(x, y, z):
        return z * 8 + y * 2 + (x if y % 2 == 0 else 1 - x)

    ring = [midx(*c) for c in cyc]
    assert sorted(ring) == list(range(N_DEV))
    inv = [0] * N_DEV
    for p, m in enumerate(ring):
        inv[m] = p
    return ring, inv

_RING, _INV = _hamiltonian_cycle()


def _ring_allreduce(y, scal):
    M, N = y.shape
    H = M // 2
    CH = H // N_DEV

    def body(scal_ref, y_ref, out_ref,
             send_a, recv_a, send_b, recv_b,
             send_sems_a, recv_sems_a, send_sems_b, recv_sems_b,
             load_sems_a, load_sems_b, store_sem_a, store_sem_b,
             credit_a, credit_b):
        pos = scal_ref[0]
        left = scal_ref[1]
        right = scal_ref[2]

        barrier_sem = pltpu.get_barrier_semaphore()
        for nbr in (left, right):
            pl.semaphore_signal(
                barrier_sem, inc=1,
                device_id=(nbr,), device_id_type=pl.DeviceIdType.MESH,
            )
        pl.semaphore_wait(barrier_sem, 2)

        def load(row0, vbuf, sem):
            cp = pltpu.make_async_copy(
                y_ref.at[pl.ds(row0, CH), :], vbuf, sem)
            cp.start()
            return cp

        def store(vbuf, row0, sem):
            cp = pltpu.make_async_copy(
                vbuf, out_ref.at[pl.ds(row0, CH), :], sem)
            cp.start()
            return cp

        def rdma_pair(g):
            slot = g % 2
            ra = pltpu.make_async_remote_copy(
                src_ref=send_a.at[slot], dst_ref=recv_a.at[slot],
                send_sem=send_sems_a.at[slot], recv_sem=recv_sems_a.at[slot],
                device_id=(right,), device_id_type=pl.DeviceIdType.MESH,
            )
            rb = pltpu.make_async_remote_copy(
                src_ref=send_b.at[slot], dst_ref=recv_b.at[slot],
                send_sem=send_sems_b.at[slot], recv_sem=recv_sems_b.at[slot],
                device_id=(left,), device_id_type=pl.DeviceIdType.MESH,
            )
            ra.start()
            rb.start()
            return ra, rb

        def credits(g):
            pl.semaphore_signal(
                credit_a, inc=1,
                device_id=(left,), device_id_type=pl.DeviceIdType.MESH,
            )
            pl.semaphore_signal(
                credit_b, inc=1,
                device_id=(right,), device_id_type=pl.DeviceIdType.MESH,
            )

        def wait_credits(g):
            if g >= 2:
                pl.semaphore_wait(credit_a, 1)
                pl.semaphore_wait(credit_b, 1)

        def rs_rows(g):
            ia = lax.rem(pos - (g + 1) + N_DEV, N_DEV)
            ib = lax.rem(pos + (g + 1), N_DEV)
            return ia * CH, H + ib * CH

        load(pos * CH, send_a.at[0], load_sems_a.at[0]).wait()
        load(H + pos * CH, send_b.at[0], load_sems_b.at[0]).wait()

        prev = None
        for g in range(N_DEV - 1):
            slot = g % 2
            nxt = (g + 1) % 2
            wait_credits(g)
            ra, rb = rdma_pair(g)
            if prev is not None:
                prev[0].wait_send()
                prev[1].wait_send()
            row_a, row_b = rs_rows(g)
            cpa = load(row_a, send_a.at[nxt], load_sems_a.at[nxt])
            cpb = load(row_b, send_b.at[nxt], load_sems_b.at[nxt])
            cpa.wait()
            cpb.wait()
            ra.wait_recv()
            send_a[nxt] = send_a[nxt] + recv_a[slot]
            rb.wait_recv()
            send_b[nxt] = send_b[nxt] + recv_b[slot]
            credits(g)
            prev = (ra, rb)

        own_a = lax.rem(pos + 1, N_DEV)
        own_b = lax.rem(pos - 1 + N_DEV, N_DEV)
        sta = store(send_a.at[1], own_a * CH, store_sem_a)
        stb = store(send_b.at[1], H + own_b * CH, store_sem_b)
        sta.wait()
        stb.wait()

        for t in range(N_DEV - 1):
            g = (N_DEV - 1) + t
            slot = g % 2
            nxt = (g + 1) % 2
            wait_credits(g)
            ra, rb = rdma_pair(g)
            prev[0].wait_send()
            prev[1].wait_send()
            ia = lax.rem(pos - t + N_DEV, N_DEV)
            ib = lax.rem(pos + t, N_DEV)
            ra.wait_recv()
            sta = store(recv_a.at[slot], ia * CH, store_sem_a)
            if t < N_DEV - 2:
                send_a[nxt] = recv_a[slot]
            rb.wait_recv()
            stb = store(recv_b.at[slot], H + ib * CH, store_sem_b)
            if t < N_DEV - 2:
                send_b[nxt] = recv_b[slot]
            sta.wait()
            stb.wait()
            credits(g)
            prev = (ra, rb)

        prev[0].wait_send()
        prev[1].wait_send()
        pl.semaphore_wait(credit_a, 2)
        pl.semaphore_wait(credit_b, 2)

    return pl.pallas_call(
        body,
        out_shape=jax.ShapeDtypeStruct((M, N), jnp.float32),
        in_specs=[
            pl.BlockSpec(memory_space=pltpu.MemorySpace.SMEM),
            pl.BlockSpec(memory_space=pl.ANY),
        ],
        out_specs=pl.BlockSpec(memory_space=pl.ANY),
        scratch_shapes=[
            pltpu.VMEM((2, CH, N), jnp.float32),
            pltpu.VMEM((2, CH, N), jnp.float32),
            pltpu.VMEM((2, CH, N), jnp.float32),
            pltpu.VMEM((2, CH, N), jnp.float32),
            pltpu.SemaphoreType.DMA((2,)),
            pltpu.SemaphoreType.DMA((2,)),
            pltpu.SemaphoreType.DMA((2,)),
            pltpu.SemaphoreType.DMA((2,)),
            pltpu.SemaphoreType.DMA((2,)),
            pltpu.SemaphoreType.DMA((2,)),
            pltpu.SemaphoreType.DMA,
            pltpu.SemaphoreType.DMA,
            pltpu.SemaphoreType.REGULAR,
            pltpu.SemaphoreType.REGULAR,
        ],
        compiler_params=pltpu.CompilerParams(collective_id=0),
    )(scal, y)


def kernel(x, w_mat):
    y = jnp.dot(x, w_mat, preferred_element_type=jnp.float32,
                precision=lax.Precision.HIGHEST)
    r = lax.axis_index("i")
    ring = jnp.asarray(_RING, jnp.int32)
    pos = jnp.asarray(_INV, jnp.int32)[r]
    right = ring[lax.rem(pos + 1, N_DEV)]
    left = ring[lax.rem(pos - 1 + N_DEV, N_DEV)]
    scal = jnp.stack([pos, left, right]).astype(jnp.int32)
    y = _ring_allreduce(y, scal)
    amax = jnp.max(jnp.abs(y))
    scale = amax / 448.0
    v = y / scale
    a = jnp.abs(v)
    _, e = jnp.frexp(a)
    q = jnp.where(a >= 2.0 ** -6,
                  jnp.ldexp(jnp.float32(1.0), e - 4),
                  jnp.float32(2.0 ** -9))
    snapped = jnp.minimum(jnp.round(a / q) * q, 448.0)
    return jnp.sign(v) * snapped * scale
